# baseline (speedup 1.0000x reference)
import functools

import jax
import jax.numpy as jnp
from jax import lax
from jax.experimental import pallas as pl
from jax.experimental.pallas import tpu as pltpu
from jax.experimental.pallas import tpu_sc as plsc

B = 8
C = 32
N = 2048
PD = 3
K = 16
TQ = 256

NC = 2
NS = 16
NW = NC * NS
ROWS = B * N * K
RPW = ROWS // NW
CHUNK = 128
NCHUNK = RPW // CHUNK


def _knn_tc_body(q_ref, rt_ref, nbr_ref):
    b = pl.program_id(0)
    q = q_ref[0]
    rt = rt_ref[0]
    chq = lax.broadcasted_iota(jnp.int32, (TQ, C), 1)
    chr_ = lax.broadcasted_iota(jnp.int32, (C, N), 0)
    qm = jnp.where(chq < PD, q, 0.0)
    rtm = jnp.where(chr_ < PD, rt, 0.0)
    cross = jnp.dot(qm, rtm, preferred_element_type=jnp.float32)
    qq = jnp.sum(qm * qm, axis=1, keepdims=True)
    rr = jnp.sum(rtm * rtm, axis=0, keepdims=True)
    d2 = qq - 2.0 * cross + rr
    iota_f = lax.broadcasted_iota(jnp.int32, (TQ, N), 1).astype(jnp.float32)
    cols = []
    for _ in range(K):
        m = jnp.min(d2, axis=1, keepdims=True)
        idxf = jnp.min(jnp.where(d2 == m, iota_f, float(N)), axis=1,
                       keepdims=True)
        cols.append(idxf)
        d2 = jnp.where(iota_f == idxf, jnp.inf, d2)
    nbr = jnp.concatenate(cols, axis=1).astype(jnp.int32) + b * N
    nbr_ref[...] = nbr


def _knn_topk(pts0r, cloud1):
    return pl.pallas_call(
        _knn_tc_body,
        grid=(B, N // TQ),
        in_specs=[
            pl.BlockSpec((1, TQ, C), lambda b, t: (b, t, 0)),
            pl.BlockSpec((1, C, N), lambda b, t: (b, 0, 0)),
        ],
        out_specs=pl.BlockSpec((TQ, K), lambda b, t: (b * (N // TQ) + t, 0)),
        out_shape=jax.ShapeDtypeStruct((B * N, K), jnp.int32),
    )(pts0r, cloud1)


def _gp0_body(q_ref, gp0_ref):
    gp0_ref[...] = jnp.broadcast_to(q_ref[...][:, None, :], (TQ, K, C))


def _gp0_write(pts0):
    return pl.pallas_call(
        _gp0_body,
        grid=(B * N // TQ,),
        in_specs=[pl.BlockSpec((TQ, C), lambda t: (t, 0))],
        out_specs=pl.BlockSpec((TQ, K, C), lambda t: (t, 0, 0)),
        out_shape=jax.ShapeDtypeStruct((B * N, K, C), jnp.float32),
    )(pts0)


@functools.cache
def _gather_sc():
    @functools.partial(
        pl.kernel,
        out_type=jax.ShapeDtypeStruct((ROWS, C), jnp.float32),
        mesh=plsc.VectorSubcoreMesh(core_axis_name="c", subcore_axis_name="s"),
        compiler_params=pltpu.CompilerParams(use_tc_tiling_on_sc=False),
        scratch_types=[
            pltpu.VMEM((CHUNK,), jnp.int32),
            pltpu.VMEM((CHUNK, C), jnp.float32),
            pltpu.SemaphoreType.DMA,
        ],
    )
    def gather(table_hbm, idx_hbm, out_hbm, idx_v, rows_v, sem):
        wid = lax.axis_index("s") * NC + lax.axis_index("c")

        def body(i, carry):
            base = wid * RPW + i * CHUNK
            pltpu.sync_copy(idx_hbm.at[pl.ds(base, CHUNK)], idx_v)
            pltpu.async_copy(table_hbm.at[idx_v], rows_v, sem).wait()
            pltpu.sync_copy(rows_v, out_hbm.at[pl.ds(base, CHUNK)])
            return carry

        lax.fori_loop(0, NCHUNK, body, 0)

    return gather


def kernel(cloud0, cloud1):
    pts0 = jnp.transpose(cloud0, (0, 2, 1)).reshape(-1, C)
    pts1 = jnp.transpose(cloud1, (0, 2, 1)).reshape(-1, C)
    nbr = _knn_topk(pts0.reshape(B, N, C), cloud1)
    gp1 = _gather_sc()(pts1, nbr.reshape(-1))
    gp0 = _gp0_write(pts0)
    return (pts0, pts1, gp0, gp1.reshape(B * N, K, C))


# speedup vs baseline: 1.1064x; 1.1064x over previous
import functools

import jax
import jax.numpy as jnp
from jax import lax
from jax.experimental import pallas as pl
from jax.experimental.pallas import tpu as pltpu
from jax.experimental.pallas import tpu_sc as plsc

B = 8
C = 32
N = 2048
PD = 3
K = 16
TQ = 256

NC = 2
NS = 16
NW = NC * NS
ROWS = B * N * K
RPW = ROWS // NW
CHUNK = 128
NCHUNK = RPW // CHUNK


def _knn_tc_body(q_ref, rt_ref, nbr_ref, gp0_ref):
    b = pl.program_id(0)
    q = q_ref[0]
    rt = rt_ref[0]
    chq = lax.broadcasted_iota(jnp.int32, (TQ, C), 1)
    chr_ = lax.broadcasted_iota(jnp.int32, (C, N), 0)
    qm = jnp.where(chq < PD, q, 0.0)
    rtm = jnp.where(chr_ < PD, rt, 0.0)
    cross = jnp.dot(qm, rtm, preferred_element_type=jnp.float32)
    qq = jnp.sum(qm * qm, axis=1, keepdims=True)
    rr = jnp.sum(rtm * rtm, axis=0, keepdims=True)
    d2 = qq - 2.0 * cross + rr
    iota_f = lax.broadcasted_iota(jnp.int32, (TQ, N), 1).astype(jnp.float32)
    cols = []
    for _ in range(K):
        m = jnp.min(d2, axis=1, keepdims=True)
        idxf = jnp.min(jnp.where(d2 == m, iota_f, float(N)), axis=1,
                       keepdims=True)
        cols.append(idxf)
        d2 = jnp.where(iota_f == idxf, jnp.inf, d2)
    nbr = jnp.concatenate(cols, axis=1).astype(jnp.int32) + b * N
    nbr_ref[...] = nbr
    gp0_ref[...] = jnp.broadcast_to(q[:, None, :], (TQ, K, C))


def _knn_topk(pts0r, cloud1):
    return pl.pallas_call(
        _knn_tc_body,
        grid=(B, N // TQ),
        in_specs=[
            pl.BlockSpec((1, TQ, C), lambda b, t: (b, t, 0)),
            pl.BlockSpec((1, C, N), lambda b, t: (b, 0, 0)),
        ],
        out_specs=[
            pl.BlockSpec((TQ, K), lambda b, t: (b * (N // TQ) + t, 0)),
            pl.BlockSpec((TQ, K, C), lambda b, t: (b * (N // TQ) + t, 0, 0)),
        ],
        out_shape=[
            jax.ShapeDtypeStruct((B * N, K), jnp.int32),
            jax.ShapeDtypeStruct((B * N, K, C), jnp.float32),
        ],
    )(pts0r, cloud1)


@functools.cache
def _gather_sc():
    @functools.partial(
        pl.kernel,
        out_type=jax.ShapeDtypeStruct((ROWS, C), jnp.float32),
        mesh=plsc.VectorSubcoreMesh(core_axis_name="c", subcore_axis_name="s"),
        compiler_params=pltpu.CompilerParams(use_tc_tiling_on_sc=False),
        scratch_types=[
            pltpu.VMEM((2, CHUNK), jnp.int32),
            pltpu.VMEM((2, CHUNK, C), jnp.float32),
            [pltpu.SemaphoreType.DMA] * 6,
        ],
    )
    def gather(table_hbm, idx_hbm, out_hbm, idx_v, rows_v, sems):
        si, sg, ss = (sems[0], sems[1]), (sems[2], sems[3]), (sems[4], sems[5])
        wid = lax.axis_index("s") * NC + lax.axis_index("c")
        first = wid * RPW

        pltpu.sync_copy(idx_hbm.at[pl.ds(first, CHUNK)], idx_v.at[0])
        pltpu.async_copy(table_hbm.at[idx_v.at[0]], rows_v.at[0], sg[0])
        pltpu.async_copy(idx_hbm.at[pl.ds(first + CHUNK, CHUNK)], idx_v.at[1],
                         si[1])
        for i in range(NCHUNK):
            a = i % 2
            if i + 1 < NCHUNK:
                pltpu.make_async_copy(
                    idx_hbm.at[pl.ds(first + (i + 1) * CHUNK, CHUNK)],
                    idx_v.at[1 - a], si[1 - a]).wait()
                if i >= 1:
                    pltpu.make_async_copy(
                        rows_v.at[1 - a],
                        out_hbm.at[pl.ds(first + (i - 1) * CHUNK, CHUNK)],
                        ss[1 - a]).wait()
                pltpu.async_copy(table_hbm.at[idx_v.at[1 - a]],
                                 rows_v.at[1 - a], sg[1 - a])
            pltpu.make_async_copy(table_hbm.at[idx_v.at[a]], rows_v.at[a],
                                  sg[a]).wait()
            if i + 2 < NCHUNK:
                pltpu.async_copy(
                    idx_hbm.at[pl.ds(first + (i + 2) * CHUNK, CHUNK)],
                    idx_v.at[a], si[a])
            pltpu.async_copy(rows_v.at[a],
                             out_hbm.at[pl.ds(first + i * CHUNK, CHUNK)],
                             ss[a])
        pltpu.make_async_copy(
            rows_v.at[0],
            out_hbm.at[pl.ds(first + (NCHUNK - 2) * CHUNK, CHUNK)],
            ss[0]).wait()
        pltpu.make_async_copy(
            rows_v.at[1],
            out_hbm.at[pl.ds(first + (NCHUNK - 1) * CHUNK, CHUNK)],
            ss[1]).wait()

    return gather


def kernel(cloud0, cloud1):
    pts0 = jnp.transpose(cloud0, (0, 2, 1)).reshape(-1, C)
    pts1 = jnp.transpose(cloud1, (0, 2, 1)).reshape(-1, C)
    nbr, gp0 = _knn_topk(pts0.reshape(B, N, C), cloud1)
    gp1 = _gather_sc()(pts1, nbr.reshape(-1))
    return (pts0, pts1, gp0, gp1.reshape(B * N, K, C))


# speedup vs baseline: 1.1073x; 1.0008x over previous
import functools

import jax
import jax.numpy as jnp
from jax import lax
from jax.experimental import pallas as pl
from jax.experimental.pallas import tpu as pltpu
from jax.experimental.pallas import tpu_sc as plsc

B = 8
C = 32
N = 2048
PD = 3
K = 16
TQ = 256

NC = 2
NS = 16
NW = NC * NS
ROWS = B * N * K
RPW = ROWS // NW
CHUNK = 128
NCHUNK = RPW // CHUNK


def _knn_tc_body(q_ref, rt_ref, nbr_ref, gp0_ref, pts0_ref, pts1_ref):
    b = pl.program_id(0)
    q = jnp.transpose(q_ref[0])
    rt = rt_ref[0]
    pts0_ref[...] = q
    pts1_ref[...] = jnp.transpose(rt)
    chq = lax.broadcasted_iota(jnp.int32, (TQ, C), 1)
    chr_ = lax.broadcasted_iota(jnp.int32, (C, N), 0)
    qm = jnp.where(chq < PD, q, 0.0)
    rtm = jnp.where(chr_ < PD, rt, 0.0)
    cross = jnp.dot(qm, rtm, preferred_element_type=jnp.float32)
    qq = jnp.sum(qm * qm, axis=1, keepdims=True)
    rr = jnp.sum(rtm * rtm, axis=0, keepdims=True)
    d2 = qq - 2.0 * cross + rr
    iota_f = lax.broadcasted_iota(jnp.int32, (TQ, N), 1).astype(jnp.float32)
    cols = []
    for _ in range(K):
        m = jnp.min(d2, axis=1, keepdims=True)
        idxf = jnp.min(jnp.where(d2 == m, iota_f, float(N)), axis=1,
                       keepdims=True)
        cols.append(idxf)
        d2 = jnp.where(iota_f == idxf, jnp.inf, d2)
    nbr = jnp.concatenate(cols, axis=1).astype(jnp.int32) + b * N
    nbr_ref[...] = nbr
    gp0_ref[...] = jnp.broadcast_to(q[:, None, :], (TQ, K, C))


def _knn_topk(cloud0, cloud1):
    return pl.pallas_call(
        _knn_tc_body,
        grid=(B, N // TQ),
        in_specs=[
            pl.BlockSpec((1, C, TQ), lambda b, t: (b, 0, t)),
            pl.BlockSpec((1, C, N), lambda b, t: (b, 0, 0)),
        ],
        out_specs=[
            pl.BlockSpec((TQ, K), lambda b, t: (b * (N // TQ) + t, 0)),
            pl.BlockSpec((TQ, K, C), lambda b, t: (b * (N // TQ) + t, 0, 0)),
            pl.BlockSpec((TQ, C), lambda b, t: (b * (N // TQ) + t, 0)),
            pl.BlockSpec((N, C), lambda b, t: (b, 0)),
        ],
        out_shape=[
            jax.ShapeDtypeStruct((B * N, K), jnp.int32),
            jax.ShapeDtypeStruct((B * N, K, C), jnp.float32),
            jax.ShapeDtypeStruct((B * N, C), jnp.float32),
            jax.ShapeDtypeStruct((B * N, C), jnp.float32),
        ],
    )(cloud0, cloud1)


@functools.cache
def _gather_sc():
    @functools.partial(
        pl.kernel,
        out_type=jax.ShapeDtypeStruct((ROWS, C), jnp.float32),
        mesh=plsc.VectorSubcoreMesh(core_axis_name="c", subcore_axis_name="s"),
        compiler_params=pltpu.CompilerParams(use_tc_tiling_on_sc=False),
        scratch_types=[
            pltpu.VMEM((2, CHUNK), jnp.int32),
            pltpu.VMEM((2, CHUNK, C), jnp.float32),
            [pltpu.SemaphoreType.DMA] * 6,
        ],
    )
    def gather(table_hbm, idx_hbm, out_hbm, idx_v, rows_v, sems):
        si, sg, ss = (sems[0], sems[1]), (sems[2], sems[3]), (sems[4], sems[5])
        wid = lax.axis_index("s") * NC + lax.axis_index("c")
        first = wid * RPW

        pltpu.sync_copy(idx_hbm.at[pl.ds(first, CHUNK)], idx_v.at[0])
        pltpu.async_copy(table_hbm.at[idx_v.at[0]], rows_v.at[0], sg[0])
        pltpu.async_copy(idx_hbm.at[pl.ds(first + CHUNK, CHUNK)], idx_v.at[1],
                         si[1])
        for i in range(NCHUNK):
            a = i % 2
            if i + 1 < NCHUNK:
                pltpu.make_async_copy(
                    idx_hbm.at[pl.ds(first + (i + 1) * CHUNK, CHUNK)],
                    idx_v.at[1 - a], si[1 - a]).wait()
                if i >= 1:
                    pltpu.make_async_copy(
                        rows_v.at[1 - a],
                        out_hbm.at[pl.ds(first + (i - 1) * CHUNK, CHUNK)],
                        ss[1 - a]).wait()
                pltpu.async_copy(table_hbm.at[idx_v.at[1 - a]],
                                 rows_v.at[1 - a], sg[1 - a])
            pltpu.make_async_copy(table_hbm.at[idx_v.at[a]], rows_v.at[a],
                                  sg[a]).wait()
            if i + 2 < NCHUNK:
                pltpu.async_copy(
                    idx_hbm.at[pl.ds(first + (i + 2) * CHUNK, CHUNK)],
                    idx_v.at[a], si[a])
            pltpu.async_copy(rows_v.at[a],
                             out_hbm.at[pl.ds(first + i * CHUNK, CHUNK)],
                             ss[a])
        pltpu.make_async_copy(
            rows_v.at[0],
            out_hbm.at[pl.ds(first + (NCHUNK - 2) * CHUNK, CHUNK)],
            ss[0]).wait()
        pltpu.make_async_copy(
            rows_v.at[1],
            out_hbm.at[pl.ds(first + (NCHUNK - 1) * CHUNK, CHUNK)],
            ss[1]).wait()

    return gather


def kernel(cloud0, cloud1):
    nbr, gp0, pts0, pts1 = _knn_topk(cloud0, cloud1)
    gp1 = _gather_sc()(pts1, nbr.reshape(-1))
    return (pts0, pts1, gp0, gp1.reshape(B * N, K, C))


# speedup vs baseline: 1.1163x; 1.0081x over previous
import functools

import jax
import jax.numpy as jnp
from jax import lax
from jax.experimental import pallas as pl
from jax.experimental.pallas import tpu as pltpu
from jax.experimental.pallas import tpu_sc as plsc

B = 8
C = 32
N = 2048
PD = 3
K = 16
TQ = 512

NC = 2
NS = 16
NW = NC * NS
ROWS = B * N * K
RPW = ROWS // NW
CHUNK = 128
NCHUNK = RPW // CHUNK


def _knn_tc_body(q_ref, rt_ref, nbr_ref, gp0_ref, pts0_ref, pts1_ref):
    b = pl.program_id(0)
    q = jnp.transpose(q_ref[0])
    rt = rt_ref[0]
    pts0_ref[...] = q
    pts1_ref[...] = jnp.transpose(rt)
    chq = lax.broadcasted_iota(jnp.int32, (TQ, C), 1)
    chr_ = lax.broadcasted_iota(jnp.int32, (C, N), 0)
    qm = jnp.where(chq < PD, q, 0.0)
    rtm = jnp.where(chr_ < PD, rt, 0.0)
    cross = jnp.dot(qm, rtm, preferred_element_type=jnp.float32)
    qq = jnp.sum(qm * qm, axis=1, keepdims=True)
    rr = jnp.sum(rtm * rtm, axis=0, keepdims=True)
    d2 = qq - 2.0 * cross + rr
    iota_f = lax.broadcasted_iota(jnp.int32, (TQ, N), 1).astype(jnp.float32)
    cols = []
    for _ in range(K):
        m = jnp.min(d2, axis=1, keepdims=True)
        idxf = jnp.min(jnp.where(d2 == m, iota_f, float(N)), axis=1,
                       keepdims=True)
        cols.append(idxf)
        d2 = jnp.where(iota_f == idxf, jnp.inf, d2)
    nbr = jnp.concatenate(cols, axis=1).astype(jnp.int32) + b * N
    nbr_ref[...] = nbr
    gp0_ref[...] = jnp.broadcast_to(q[:, None, :], (TQ, K, C))


def _knn_topk(cloud0, cloud1):
    return pl.pallas_call(
        _knn_tc_body,
        grid=(B, N // TQ),
        in_specs=[
            pl.BlockSpec((1, C, TQ), lambda b, t: (b, 0, t)),
            pl.BlockSpec((1, C, N), lambda b, t: (b, 0, 0)),
        ],
        out_specs=[
            pl.BlockSpec((TQ, K), lambda b, t: (b * (N // TQ) + t, 0)),
            pl.BlockSpec((TQ, K, C), lambda b, t: (b * (N // TQ) + t, 0, 0)),
            pl.BlockSpec((TQ, C), lambda b, t: (b * (N // TQ) + t, 0)),
            pl.BlockSpec((N, C), lambda b, t: (b, 0)),
        ],
        out_shape=[
            jax.ShapeDtypeStruct((B * N, K), jnp.int32),
            jax.ShapeDtypeStruct((B * N, K, C), jnp.float32),
            jax.ShapeDtypeStruct((B * N, C), jnp.float32),
            jax.ShapeDtypeStruct((B * N, C), jnp.float32),
        ],
    )(cloud0, cloud1)


@functools.cache
def _gather_sc():
    @functools.partial(
        pl.kernel,
        out_type=jax.ShapeDtypeStruct((ROWS, C), jnp.float32),
        mesh=plsc.VectorSubcoreMesh(core_axis_name="c", subcore_axis_name="s"),
        compiler_params=pltpu.CompilerParams(use_tc_tiling_on_sc=False),
        scratch_types=[
            pltpu.VMEM((2, CHUNK), jnp.int32),
            pltpu.VMEM((2, CHUNK, C), jnp.float32),
            [pltpu.SemaphoreType.DMA] * 6,
        ],
    )
    def gather(table_hbm, idx_hbm, out_hbm, idx_v, rows_v, sems):
        si, sg, ss = (sems[0], sems[1]), (sems[2], sems[3]), (sems[4], sems[5])
        wid = lax.axis_index("s") * NC + lax.axis_index("c")
        first = wid * RPW

        pltpu.sync_copy(idx_hbm.at[pl.ds(first, CHUNK)], idx_v.at[0])
        pltpu.async_copy(table_hbm.at[idx_v.at[0]], rows_v.at[0], sg[0])
        pltpu.async_copy(idx_hbm.at[pl.ds(first + CHUNK, CHUNK)], idx_v.at[1],
                         si[1])
        for i in range(NCHUNK):
            a = i % 2
            if i + 1 < NCHUNK:
                pltpu.make_async_copy(
                    idx_hbm.at[pl.ds(first + (i + 1) * CHUNK, CHUNK)],
                    idx_v.at[1 - a], si[1 - a]).wait()
                if i >= 1:
                    pltpu.make_async_copy(
                        rows_v.at[1 - a],
                        out_hbm.at[pl.ds(first + (i - 1) * CHUNK, CHUNK)],
                        ss[1 - a]).wait()
                pltpu.async_copy(table_hbm.at[idx_v.at[1 - a]],
                                 rows_v.at[1 - a], sg[1 - a])
            pltpu.make_async_copy(table_hbm.at[idx_v.at[a]], rows_v.at[a],
                                  sg[a]).wait()
            if i + 2 < NCHUNK:
                pltpu.async_copy(
                    idx_hbm.at[pl.ds(first + (i + 2) * CHUNK, CHUNK)],
                    idx_v.at[a], si[a])
            pltpu.async_copy(rows_v.at[a],
                             out_hbm.at[pl.ds(first + i * CHUNK, CHUNK)],
                             ss[a])
        pltpu.make_async_copy(
            rows_v.at[0],
            out_hbm.at[pl.ds(first + (NCHUNK - 2) * CHUNK, CHUNK)],
            ss[0]).wait()
        pltpu.make_async_copy(
            rows_v.at[1],
            out_hbm.at[pl.ds(first + (NCHUNK - 1) * CHUNK, CHUNK)],
            ss[1]).wait()

    return gather


def kernel(cloud0, cloud1):
    nbr, gp0, pts0, pts1 = _knn_topk(cloud0, cloud1)
    gp1 = _gather_sc()(pts1, nbr.reshape(-1))
    return (pts0, pts1, gp0, gp1.reshape(B * N, K, C))
